# baseline (speedup 1.0000x reference)
import functools

import jax
import jax.numpy as jnp
from jax import lax
from jax.experimental import pallas as pl
from jax.experimental.pallas import tpu as pltpu
from jax.experimental.pallas import tpu_sc as plsc

NC = 2
NS = 16
NW = NC * NS
LANES = 16

DIM = 32
ROWW = 128
PACK = ROWW // DIM
BATCH = 16384
TOT = 2 * BATCH
BPW = TOT // NW
CHUNK = 128
NCHUNK = BPW // CHUNK
GRP = CHUNK // LANES


def _tpu_kernel(e4, r4, h_idx4, r_idx4, t_idx4, h_sub, r_sub, t_sub):
    mesh = plsc.VectorSubcoreMesh(core_axis_name="c", subcore_axis_name="s")

    @functools.partial(
        pl.kernel,
        out_type=jax.ShapeDtypeStruct((TOT,), jnp.float32),
        mesh=mesh,
        compiler_params=pltpu.CompilerParams(needs_layout_passes=False),
        scratch_types=dict(
            hi=pltpu.VMEM((NCHUNK, CHUNK), jnp.int32),
            ri=pltpu.VMEM((NCHUNK, CHUNK), jnp.int32),
            ti=pltpu.VMEM((NCHUNK, CHUNK), jnp.int32),
            hs=pltpu.VMEM((BPW,), jnp.int32),
            rs=pltpu.VMEM((BPW,), jnp.int32),
            ts=pltpu.VMEM((BPW,), jnp.int32),
            hrow=pltpu.VMEM((CHUNK, ROWW), jnp.float32),
            rrow=pltpu.VMEM((CHUNK, ROWW), jnp.float32),
            trow=pltpu.VMEM((CHUNK, ROWW), jnp.float32),
            out_v=pltpu.VMEM((BPW,), jnp.float32),
            sem_h=pltpu.SemaphoreType.DMA,
            sem_r=pltpu.SemaphoreType.DMA,
            sem_t=pltpu.SemaphoreType.DMA,
        ),
    )
    def run(e_hbm, rel_hbm, hi_hbm, ri_hbm, ti_hbm, hs_hbm, rs_hbm, ts_hbm,
            out_hbm, hi, ri, ti, hs, rs, ts, hrow, rrow, trow, out_v,
            sem_h, sem_r, sem_t):
        wid = lax.axis_index("s") * NC + lax.axis_index("c")
        pltpu.sync_copy(hi_hbm.at[wid], hi)
        pltpu.sync_copy(ri_hbm.at[wid], ri)
        pltpu.sync_copy(ti_hbm.at[wid], ti)
        pltpu.sync_copy(hs_hbm.at[pl.ds(wid * BPW, BPW)], hs)
        pltpu.sync_copy(rs_hbm.at[pl.ds(wid * BPW, BPW)], rs)
        pltpu.sync_copy(ts_hbm.at[pl.ds(wid * BPW, BPW)], ts)

        def chunk_body(c, carry):
            ch = pltpu.async_copy(e_hbm.at[hi.at[c]], hrow, sem_h)
            cr = pltpu.async_copy(rel_hbm.at[ri.at[c]], rrow, sem_r)
            ct = pltpu.async_copy(e_hbm.at[ti.at[c]], trow, sem_t)
            ch.wait()
            cr.wait()
            ct.wait()

            def grp(g, carry2):
                base = c * CHUNK + g * LANES
                rows = lax.iota(jnp.int32, LANES) + g * LANES
                sh = hs[pl.ds(base, LANES)]
                sr = rs[pl.ds(base, LANES)]
                st = ts[pl.ds(base, LANES)]
                acc = jnp.zeros((LANES,), jnp.float32)
                for d in range(DIM):
                    hc = plsc.load_gather(hrow, [rows, sh + d])
                    rc = plsc.load_gather(rrow, [rows, sr + d])
                    tc = plsc.load_gather(trow, [rows, st + d])
                    acc = acc + jnp.abs(hc + rc - tc)
                out_v[pl.ds(base, LANES)] = acc
                return carry2

            lax.fori_loop(0, GRP, grp, 0)
            return carry

        lax.fori_loop(0, NCHUNK, chunk_body, 0)
        pltpu.sync_copy(out_v, out_hbm.at[pl.ds(wid * BPW, BPW)])

    return run(e4, r4, h_idx4, r_idx4, t_idx4, h_sub, r_sub, t_sub)


def kernel(pos_triples, neg_triples, e_weight, r_weight):
    trip = jnp.concatenate(
        [pos_triples.astype(jnp.int32), neg_triples.astype(jnp.int32)], axis=1)
    e4 = e_weight.reshape(-1, ROWW)
    r4 = r_weight.reshape(-1, ROWW)
    idx4 = jnp.right_shift(trip, 2).reshape(3, NW, NCHUNK, CHUNK)
    sub = (jnp.bitwise_and(trip, 3) * DIM).reshape(3, NW * BPW)
    out = _tpu_kernel(e4, r4, idx4[0], idx4[1], idx4[2],
                      sub[0], sub[1], sub[2])
    return (out[:BATCH], out[BATCH:])


# speedup vs baseline: 1.6145x; 1.6145x over previous
import functools

import jax
import jax.numpy as jnp
from jax import lax
from jax.experimental import pallas as pl
from jax.experimental.pallas import tpu as pltpu
from jax.experimental.pallas import tpu_sc as plsc

NC = 2
NS = 16
NW = NC * NS
LANES = 16

DIM = 32
ROWW = 128
PACK = ROWW // DIM
V_E = 1000000
BATCH = 16384
TOT = 2 * BATCH
BPW = TOT // NW
CHUNK = 128
NCHUNK = BPW // CHUNK
GRP = CHUNK // LANES

TBLK = 8192
SUB = TBLK // PACK
TSTEPS = -(-V_E // TBLK)
VROWS = TSTEPS * SUB


def _pack_rows_tc(xT):
    def body(x_ref, o_ref):
        x = x_ref[...]
        for s in range(PACK):
            o_ref[:, pl.ds(s * DIM, DIM)] = x[:, s * SUB:(s + 1) * SUB].T

    return pl.pallas_call(
        body,
        grid=(TSTEPS,),
        in_specs=[pl.BlockSpec((DIM, TBLK), lambda j: (0, j))],
        out_specs=pl.BlockSpec((SUB, ROWW), lambda j: (j, 0)),
        out_shape=jax.ShapeDtypeStruct((VROWS, ROWW), jnp.float32),
    )(xT)


def _gather_sc(e4, r4, h_idx4, r_idx4, t_idx4, h_sub, r_sub, t_sub):
    mesh = plsc.VectorSubcoreMesh(core_axis_name="c", subcore_axis_name="s")

    @functools.partial(
        pl.kernel,
        out_type=jax.ShapeDtypeStruct((TOT,), jnp.float32),
        mesh=mesh,
        compiler_params=pltpu.CompilerParams(needs_layout_passes=False),
        scratch_types=dict(
            hi=pltpu.VMEM((NCHUNK, CHUNK), jnp.int32),
            ri=pltpu.VMEM((NCHUNK, CHUNK), jnp.int32),
            ti=pltpu.VMEM((NCHUNK, CHUNK), jnp.int32),
            hs=pltpu.VMEM((BPW,), jnp.int32),
            rs=pltpu.VMEM((BPW,), jnp.int32),
            ts=pltpu.VMEM((BPW,), jnp.int32),
            hrow=pltpu.VMEM((CHUNK, ROWW), jnp.float32),
            rrow=pltpu.VMEM((CHUNK, ROWW), jnp.float32),
            trow=pltpu.VMEM((CHUNK, ROWW), jnp.float32),
            out_v=pltpu.VMEM((BPW,), jnp.float32),
            sem_h=pltpu.SemaphoreType.DMA,
            sem_r=pltpu.SemaphoreType.DMA,
            sem_t=pltpu.SemaphoreType.DMA,
        ),
    )
    def run(e_hbm, rel_hbm, hi_hbm, ri_hbm, ti_hbm, hs_hbm, rs_hbm, ts_hbm,
            out_hbm, hi, ri, ti, hs, rs, ts, hrow, rrow, trow, out_v,
            sem_h, sem_r, sem_t):
        wid = lax.axis_index("s") * NC + lax.axis_index("c")
        pltpu.sync_copy(hi_hbm.at[wid], hi)
        pltpu.sync_copy(ri_hbm.at[wid], ri)
        pltpu.sync_copy(ti_hbm.at[wid], ti)
        pltpu.sync_copy(hs_hbm.at[pl.ds(wid * BPW, BPW)], hs)
        pltpu.sync_copy(rs_hbm.at[pl.ds(wid * BPW, BPW)], rs)
        pltpu.sync_copy(ts_hbm.at[pl.ds(wid * BPW, BPW)], ts)

        def chunk_body(c, carry):
            ch = pltpu.async_copy(e_hbm.at[hi.at[c]], hrow, sem_h)
            cr = pltpu.async_copy(rel_hbm.at[ri.at[c]], rrow, sem_r)
            ct = pltpu.async_copy(e_hbm.at[ti.at[c]], trow, sem_t)
            ch.wait()
            cr.wait()
            ct.wait()

            def grp(g, carry2):
                base = c * CHUNK + g * LANES
                rows = lax.iota(jnp.int32, LANES) + g * LANES
                sh = hs[pl.ds(base, LANES)]
                sr = rs[pl.ds(base, LANES)]
                st = ts[pl.ds(base, LANES)]
                acc = jnp.zeros((LANES,), jnp.float32)
                for d in range(DIM):
                    hc = plsc.load_gather(hrow, [rows, sh + d])
                    rc = plsc.load_gather(rrow, [rows, sr + d])
                    tc = plsc.load_gather(trow, [rows, st + d])
                    acc = acc + jnp.abs(hc + rc - tc)
                out_v[pl.ds(base, LANES)] = acc
                return carry2

            lax.fori_loop(0, GRP, grp, 0)
            return carry

        lax.fori_loop(0, NCHUNK, chunk_body, 0)
        pltpu.sync_copy(out_v, out_hbm.at[pl.ds(wid * BPW, BPW)])

    return run(e4, r4, h_idx4, r_idx4, t_idx4, h_sub, r_sub, t_sub)


def kernel(pos_triples, neg_triples, e_weight, r_weight):
    e4 = _pack_rows_tc(e_weight.T)
    r4 = _pack_rows_tc(r_weight.T)
    trip = jnp.concatenate(
        [pos_triples.astype(jnp.int32), neg_triples.astype(jnp.int32)], axis=1)
    row = (trip // TBLK) * SUB + jnp.remainder(trip, SUB)
    idx4 = row.reshape(3, NW, NCHUNK, CHUNK)
    sub = (jnp.remainder(trip // SUB, PACK) * DIM).reshape(3, NW * BPW)
    out = _gather_sc(e4, r4, idx4[0], idx4[1], idx4[2],
                     sub[0], sub[1], sub[2])
    return (out[:BATCH], out[BATCH:])


# speedup vs baseline: 2.1282x; 1.3182x over previous
import functools

import jax
import jax.numpy as jnp
from jax import lax
from jax.experimental import pallas as pl
from jax.experimental.pallas import tpu as pltpu
from jax.experimental.pallas import tpu_sc as plsc

NC = 2
NS = 16
NW = NC * NS
LANES = 16

DIM = 32
ROWW = 128
PACK = ROWW // DIM
V_E = 1000000
BATCH = 16384
TOT = 2 * BATCH
BPW = TOT // NW
CHUNK = 128
NCHUNK = BPW // CHUNK
GRP = CHUNK // LANES

TBLK = 8192
SUB = TBLK // PACK
TSTEPS = -(-V_E // TBLK)
VROWS = TSTEPS * SUB


def _pack_rows_tc(xT):
    def body(x_ref, o_ref):
        x = x_ref[...]
        xx = jnp.concatenate(
            [x[:, s * SUB:(s + 1) * SUB] for s in range(PACK)], axis=0)
        eye = jnp.eye(ROWW, dtype=jnp.float32)
        o_ref[...] = lax.dot_general(xx, eye, (((0,), (0,)), ((), ())),
                                     precision=lax.Precision.HIGHEST,
                                     preferred_element_type=jnp.float32)

    return pl.pallas_call(
        body,
        grid=(TSTEPS,),
        in_specs=[pl.BlockSpec((DIM, TBLK), lambda j: (0, j))],
        out_specs=pl.BlockSpec((SUB, ROWW), lambda j: (j, 0)),
        out_shape=jax.ShapeDtypeStruct((VROWS, ROWW), jnp.float32),
    )(xT)


def _gather_sc(e4, r4, h_idx4, r_idx4, t_idx4, h_sub, r_sub, t_sub):
    mesh = plsc.VectorSubcoreMesh(core_axis_name="c", subcore_axis_name="s")

    @functools.partial(
        pl.kernel,
        out_type=jax.ShapeDtypeStruct((TOT,), jnp.float32),
        mesh=mesh,
        compiler_params=pltpu.CompilerParams(needs_layout_passes=False),
        scratch_types=dict(
            hi=pltpu.VMEM((NCHUNK, CHUNK), jnp.int32),
            ri=pltpu.VMEM((NCHUNK, CHUNK), jnp.int32),
            ti=pltpu.VMEM((NCHUNK, CHUNK), jnp.int32),
            hs=pltpu.VMEM((BPW,), jnp.int32),
            rs=pltpu.VMEM((BPW,), jnp.int32),
            ts=pltpu.VMEM((BPW,), jnp.int32),
            hrow=pltpu.VMEM((CHUNK, ROWW), jnp.float32),
            rrow=pltpu.VMEM((CHUNK, ROWW), jnp.float32),
            trow=pltpu.VMEM((CHUNK, ROWW), jnp.float32),
            out_v=pltpu.VMEM((BPW,), jnp.float32),
            sem_h=pltpu.SemaphoreType.DMA,
            sem_r=pltpu.SemaphoreType.DMA,
            sem_t=pltpu.SemaphoreType.DMA,
        ),
    )
    def run(e_hbm, rel_hbm, hi_hbm, ri_hbm, ti_hbm, hs_hbm, rs_hbm, ts_hbm,
            out_hbm, hi, ri, ti, hs, rs, ts, hrow, rrow, trow, out_v,
            sem_h, sem_r, sem_t):
        wid = lax.axis_index("s") * NC + lax.axis_index("c")
        pltpu.sync_copy(hi_hbm.at[wid], hi)
        pltpu.sync_copy(ri_hbm.at[wid], ri)
        pltpu.sync_copy(ti_hbm.at[wid], ti)
        pltpu.sync_copy(hs_hbm.at[pl.ds(wid * BPW, BPW)], hs)
        pltpu.sync_copy(rs_hbm.at[pl.ds(wid * BPW, BPW)], rs)
        pltpu.sync_copy(ts_hbm.at[pl.ds(wid * BPW, BPW)], ts)

        def chunk_body(c, carry):
            ch = pltpu.async_copy(e_hbm.at[hi.at[c]], hrow, sem_h)
            cr = pltpu.async_copy(rel_hbm.at[ri.at[c]], rrow, sem_r)
            ct = pltpu.async_copy(e_hbm.at[ti.at[c]], trow, sem_t)
            ch.wait()
            cr.wait()
            ct.wait()

            def grp(g, carry2):
                base = c * CHUNK + g * LANES
                rows = lax.iota(jnp.int32, LANES) + g * LANES
                sh = hs[pl.ds(base, LANES)]
                sr = rs[pl.ds(base, LANES)]
                st = ts[pl.ds(base, LANES)]
                acc = jnp.zeros((LANES,), jnp.float32)
                for d in range(DIM):
                    hc = plsc.load_gather(hrow, [rows, sh + d])
                    rc = plsc.load_gather(rrow, [rows, sr + d])
                    tc = plsc.load_gather(trow, [rows, st + d])
                    acc = acc + jnp.abs(hc + rc - tc)
                out_v[pl.ds(base, LANES)] = acc
                return carry2

            lax.fori_loop(0, GRP, grp, 0)
            return carry

        lax.fori_loop(0, NCHUNK, chunk_body, 0)
        pltpu.sync_copy(out_v, out_hbm.at[pl.ds(wid * BPW, BPW)])

    return run(e4, r4, h_idx4, r_idx4, t_idx4, h_sub, r_sub, t_sub)


def kernel(pos_triples, neg_triples, e_weight, r_weight):
    e4 = _pack_rows_tc(e_weight.T)
    r4 = _pack_rows_tc(r_weight.T)
    trip = jnp.concatenate(
        [pos_triples.astype(jnp.int32), neg_triples.astype(jnp.int32)], axis=1)
    row = (trip // TBLK) * SUB + jnp.remainder(trip, SUB)
    idx4 = row.reshape(3, NW, NCHUNK, CHUNK)
    sub = (jnp.remainder(trip // SUB, PACK) * DIM).reshape(3, NW * BPW)
    out = _gather_sc(e4, r4, idx4[0], idx4[1], idx4[2],
                     sub[0], sub[1], sub[2])
    return (out[:BATCH], out[BATCH:])


# speedup vs baseline: 2.2229x; 1.0445x over previous
import functools

import jax
import jax.numpy as jnp
from jax import lax
from jax.experimental import pallas as pl
from jax.experimental.pallas import tpu as pltpu
from jax.experimental.pallas import tpu_sc as plsc

NC = 2
NS = 16
NW = NC * NS
LANES = 16

DIM = 32
ROWW = 128
PACK = ROWW // DIM
V_E = 1000000
BATCH = 16384
TOT = 2 * BATCH
BPW = TOT // NW
CHUNK = 128
NCHUNK = BPW // CHUNK
GRP = CHUNK // LANES

TBLK = 8192
SUB = TBLK // PACK
TSTEPS = -(-V_E // TBLK)
VROWS = TSTEPS * SUB


def _pack_rows_tc(xT):
    def body(x_ref, o_ref):
        x = x_ref[...]
        xx = jnp.concatenate(
            [x[:, s * SUB:(s + 1) * SUB] for s in range(PACK)], axis=0)
        eye = jnp.eye(ROWW, dtype=jnp.float32)
        o_ref[...] = lax.dot_general(xx, eye, (((0,), (0,)), ((), ())),
                                     precision=lax.Precision.HIGHEST,
                                     preferred_element_type=jnp.float32)

    return pl.pallas_call(
        body,
        grid=(TSTEPS,),
        in_specs=[pl.BlockSpec((DIM, TBLK), lambda j: (0, j))],
        out_specs=pl.BlockSpec((SUB, ROWW), lambda j: (j, 0)),
        out_shape=jax.ShapeDtypeStruct((VROWS, ROWW), jnp.float32),
    )(xT)


def _gather_sc(e4, r4, h_idx4, r_idx4, t_idx4, h_sub, r_sub, t_sub):
    mesh = plsc.VectorSubcoreMesh(core_axis_name="c", subcore_axis_name="s")

    @functools.partial(
        pl.kernel,
        out_type=jax.ShapeDtypeStruct((TOT,), jnp.float32),
        mesh=mesh,
        compiler_params=pltpu.CompilerParams(needs_layout_passes=False),
        scratch_types=dict(
            hi=pltpu.VMEM((NCHUNK, CHUNK), jnp.int32),
            ri=pltpu.VMEM((NCHUNK, CHUNK), jnp.int32),
            ti=pltpu.VMEM((NCHUNK, CHUNK), jnp.int32),
            hs=pltpu.VMEM((BPW,), jnp.int32),
            rs=pltpu.VMEM((BPW,), jnp.int32),
            ts=pltpu.VMEM((BPW,), jnp.int32),
            hrow=pltpu.VMEM((2, CHUNK, ROWW), jnp.float32),
            rrow=pltpu.VMEM((2, CHUNK, ROWW), jnp.float32),
            trow=pltpu.VMEM((2, CHUNK, ROWW), jnp.float32),
            out_v=pltpu.VMEM((BPW,), jnp.float32),
            sem_h=pltpu.SemaphoreType.DMA((2,)),
            sem_r=pltpu.SemaphoreType.DMA((2,)),
            sem_t=pltpu.SemaphoreType.DMA((2,)),
        ),
    )
    def run(e_hbm, rel_hbm, hi_hbm, ri_hbm, ti_hbm, hs_hbm, rs_hbm, ts_hbm,
            out_hbm, hi, ri, ti, hs, rs, ts, hrow, rrow, trow, out_v,
            sem_h, sem_r, sem_t):
        wid = lax.axis_index("s") * NC + lax.axis_index("c")
        pltpu.sync_copy(hi_hbm.at[wid], hi)
        pltpu.sync_copy(ri_hbm.at[wid], ri)
        pltpu.sync_copy(ti_hbm.at[wid], ti)
        pltpu.sync_copy(hs_hbm.at[pl.ds(wid * BPW, BPW)], hs)
        pltpu.sync_copy(rs_hbm.at[pl.ds(wid * BPW, BPW)], rs)
        pltpu.sync_copy(ts_hbm.at[pl.ds(wid * BPW, BPW)], ts)

        def fire(c, b):
            pltpu.async_copy(e_hbm.at[hi.at[c]], hrow.at[b], sem_h.at[b])
            pltpu.async_copy(rel_hbm.at[ri.at[c]], rrow.at[b], sem_r.at[b])
            pltpu.async_copy(e_hbm.at[ti.at[c]], trow.at[b], sem_t.at[b])

        fire(0, 0)

        def chunk_body(c, carry):
            b = lax.rem(c, 2)

            @pl.when(c + 1 < NCHUNK)
            def _():
                fire(c + 1, 1 - b)

            dummy = e_hbm.at[pl.ds(0, CHUNK)]
            pltpu.make_async_copy(dummy, hrow.at[b], sem_h.at[b]).wait()
            pltpu.make_async_copy(dummy, rrow.at[b], sem_r.at[b]).wait()
            pltpu.make_async_copy(dummy, trow.at[b], sem_t.at[b]).wait()

            def grp(g, carry2):
                base = c * CHUNK + g * LANES
                rows = lax.iota(jnp.int32, LANES) + g * LANES
                sh = hs[pl.ds(base, LANES)]
                sr = rs[pl.ds(base, LANES)]
                st = ts[pl.ds(base, LANES)]
                acc = jnp.zeros((LANES,), jnp.float32)
                for d in range(DIM):
                    hc = plsc.load_gather(hrow.at[b], [rows, sh + d])
                    rc = plsc.load_gather(rrow.at[b], [rows, sr + d])
                    tc = plsc.load_gather(trow.at[b], [rows, st + d])
                    acc = acc + jnp.abs(hc + rc - tc)
                out_v[pl.ds(base, LANES)] = acc
                return carry2

            lax.fori_loop(0, GRP, grp, 0)
            return carry

        lax.fori_loop(0, NCHUNK, chunk_body, 0)
        pltpu.sync_copy(out_v, out_hbm.at[pl.ds(wid * BPW, BPW)])

    return run(e4, r4, h_idx4, r_idx4, t_idx4, h_sub, r_sub, t_sub)


def kernel(pos_triples, neg_triples, e_weight, r_weight):
    e4 = _pack_rows_tc(e_weight.T)
    r4 = _pack_rows_tc(r_weight.T)
    trip = jnp.concatenate(
        [pos_triples.astype(jnp.int32), neg_triples.astype(jnp.int32)], axis=1)
    row = (trip // TBLK) * SUB + jnp.remainder(trip, SUB)
    idx4 = row.reshape(3, NW, NCHUNK, CHUNK)
    sub = (jnp.remainder(trip // SUB, PACK) * DIM).reshape(3, NW * BPW)
    out = _gather_sc(e4, r4, idx4[0], idx4[1], idx4[2],
                     sub[0], sub[1], sub[2])
    return (out[:BATCH], out[BATCH:])


# speedup vs baseline: 2.6766x; 1.2041x over previous
import functools

import jax
import jax.numpy as jnp
from jax import lax
from jax.experimental import pallas as pl
from jax.experimental.pallas import tpu as pltpu
from jax.experimental.pallas import tpu_sc as plsc

NC = 2
NS = 16
NW = NC * NS
LANES = 16

DIM = 32
ROWW = 128
PACK = ROWW // DIM
V_E = 1000000
BATCH = 16384
TOT = 2 * BATCH
BPW = TOT // NW
CHUNK = 128
NCHUNK = BPW // CHUNK
GRP = CHUNK // LANES

TBLK = 16384
SUB = TBLK // PACK
TSTEPS = -(-V_E // TBLK)
VROWS = TSTEPS * SUB


def _pack_rows_tc(xT):
    def body(x_ref, o_ref):
        x = x_ref[...]
        xx = jnp.concatenate(
            [x[:, s * SUB:(s + 1) * SUB] for s in range(PACK)], axis=0)
        eye = jnp.eye(ROWW, dtype=jnp.float32)
        o_ref[...] = lax.dot_general(xx, eye, (((0,), (0,)), ((), ())),
                                     precision=lax.Precision.HIGHEST,
                                     preferred_element_type=jnp.float32)

    return pl.pallas_call(
        body,
        grid=(TSTEPS,),
        in_specs=[pl.BlockSpec((DIM, TBLK), lambda j: (0, j))],
        out_specs=pl.BlockSpec((SUB, ROWW), lambda j: (j, 0)),
        out_shape=jax.ShapeDtypeStruct((VROWS, ROWW), jnp.float32),
    )(xT)


def _gather_sc(e4, r4, h_idx4, r_idx4, t_idx4, h_sub, r_sub, t_sub):
    mesh = plsc.VectorSubcoreMesh(core_axis_name="c", subcore_axis_name="s")

    @functools.partial(
        pl.kernel,
        out_type=jax.ShapeDtypeStruct((TOT,), jnp.float32),
        mesh=mesh,
        compiler_params=pltpu.CompilerParams(needs_layout_passes=False),
        scratch_types=dict(
            hi=pltpu.VMEM((NCHUNK, CHUNK), jnp.int32),
            ri=pltpu.VMEM((NCHUNK, CHUNK), jnp.int32),
            ti=pltpu.VMEM((NCHUNK, CHUNK), jnp.int32),
            hs=pltpu.VMEM((BPW,), jnp.int32),
            rs=pltpu.VMEM((BPW,), jnp.int32),
            ts=pltpu.VMEM((BPW,), jnp.int32),
            hrow=pltpu.VMEM((2, CHUNK, ROWW), jnp.float32),
            rrow=pltpu.VMEM((2, CHUNK, ROWW), jnp.float32),
            trow=pltpu.VMEM((2, CHUNK, ROWW), jnp.float32),
            out_v=pltpu.VMEM((BPW,), jnp.float32),
            sem_h=pltpu.SemaphoreType.DMA((2,)),
            sem_r=pltpu.SemaphoreType.DMA((2,)),
            sem_t=pltpu.SemaphoreType.DMA((2,)),
        ),
    )
    def run(e_hbm, rel_hbm, hi_hbm, ri_hbm, ti_hbm, hs_hbm, rs_hbm, ts_hbm,
            out_hbm, hi, ri, ti, hs, rs, ts, hrow, rrow, trow, out_v,
            sem_h, sem_r, sem_t):
        wid = lax.axis_index("s") * NC + lax.axis_index("c")
        pltpu.sync_copy(hi_hbm.at[wid], hi)
        pltpu.sync_copy(ri_hbm.at[wid], ri)
        pltpu.sync_copy(ti_hbm.at[wid], ti)
        pltpu.sync_copy(hs_hbm.at[pl.ds(wid * BPW, BPW)], hs)
        pltpu.sync_copy(rs_hbm.at[pl.ds(wid * BPW, BPW)], rs)
        pltpu.sync_copy(ts_hbm.at[pl.ds(wid * BPW, BPW)], ts)

        def fire(c, b):
            pltpu.async_copy(e_hbm.at[hi.at[c]], hrow.at[b], sem_h.at[b])
            pltpu.async_copy(rel_hbm.at[ri.at[c]], rrow.at[b], sem_r.at[b])
            pltpu.async_copy(e_hbm.at[ti.at[c]], trow.at[b], sem_t.at[b])

        fire(0, 0)

        def chunk_body(c, carry):
            b = lax.rem(c, 2)

            @pl.when(c + 1 < NCHUNK)
            def _():
                fire(c + 1, 1 - b)

            dummy = e_hbm.at[pl.ds(0, CHUNK)]
            pltpu.make_async_copy(dummy, hrow.at[b], sem_h.at[b]).wait()
            pltpu.make_async_copy(dummy, rrow.at[b], sem_r.at[b]).wait()
            pltpu.make_async_copy(dummy, trow.at[b], sem_t.at[b]).wait()

            def grp(g, carry2):
                base = c * CHUNK + g * LANES
                rows = lax.iota(jnp.int32, LANES) + g * LANES
                sh = hs[pl.ds(base, LANES)]
                sr = rs[pl.ds(base, LANES)]
                st = ts[pl.ds(base, LANES)]
                acc = jnp.zeros((LANES,), jnp.float32)
                for d in range(DIM):
                    hc = plsc.load_gather(hrow.at[b], [rows, sh + d])
                    rc = plsc.load_gather(rrow.at[b], [rows, sr + d])
                    tc = plsc.load_gather(trow.at[b], [rows, st + d])
                    acc = acc + jnp.abs(hc + rc - tc)
                out_v[pl.ds(base, LANES)] = acc
                return carry2

            lax.fori_loop(0, GRP, grp, 0)
            return carry

        lax.fori_loop(0, NCHUNK, chunk_body, 0)
        pltpu.sync_copy(out_v, out_hbm.at[pl.ds(wid * BPW, BPW)])

    return run(e4, r4, h_idx4, r_idx4, t_idx4, h_sub, r_sub, t_sub)


def kernel(pos_triples, neg_triples, e_weight, r_weight):
    e4 = _pack_rows_tc(e_weight.T)
    r4 = _pack_rows_tc(r_weight.T)
    trip = jnp.concatenate(
        [pos_triples.astype(jnp.int32), neg_triples.astype(jnp.int32)], axis=1)
    row = (trip // TBLK) * SUB + jnp.remainder(trip, SUB)
    idx4 = row.reshape(3, NW, NCHUNK, CHUNK)
    sub = (jnp.remainder(trip // SUB, PACK) * DIM).reshape(3, NW * BPW)
    out = _gather_sc(e4, r4, idx4[0], idx4[1], idx4[2],
                     sub[0], sub[1], sub[2])
    return (out[:BATCH], out[BATCH:])


# speedup vs baseline: 2.9935x; 1.1184x over previous
import functools

import jax
import jax.numpy as jnp
from jax import lax
from jax.experimental import pallas as pl
from jax.experimental.pallas import tpu as pltpu
from jax.experimental.pallas import tpu_sc as plsc

NC = 2
NS = 16
NW = NC * NS
LANES = 16

DIM = 32
ROWW = 128
PACK = ROWW // DIM
V_E = 1000000
BATCH = 16384
TOT = 2 * BATCH
BPW = TOT // NW
CHUNK = 128
NCHUNK = BPW // CHUNK
GRP = CHUNK // LANES

TBLK = 32768
SUB = TBLK // PACK
TSTEPS = -(-V_E // TBLK)
VROWS = TSTEPS * SUB


def _pack_rows_tc(xT):
    def body(x_ref, o_ref):
        x = x_ref[...]
        xx = jnp.concatenate(
            [x[:, s * SUB:(s + 1) * SUB] for s in range(PACK)], axis=0)
        eye = jnp.eye(ROWW, dtype=jnp.float32)
        o_ref[...] = lax.dot_general(xx, eye, (((0,), (0,)), ((), ())),
                                     precision=lax.Precision.HIGHEST,
                                     preferred_element_type=jnp.float32)

    return pl.pallas_call(
        body,
        grid=(TSTEPS,),
        in_specs=[pl.BlockSpec((DIM, TBLK), lambda j: (0, j))],
        out_specs=pl.BlockSpec((SUB, ROWW), lambda j: (j, 0)),
        out_shape=jax.ShapeDtypeStruct((VROWS, ROWW), jnp.float32),
    )(xT)


def _gather_sc(e4, r4, h_idx4, r_idx4, t_idx4, h_sub, r_sub, t_sub):
    mesh = plsc.VectorSubcoreMesh(core_axis_name="c", subcore_axis_name="s")

    @functools.partial(
        pl.kernel,
        out_type=jax.ShapeDtypeStruct((TOT,), jnp.float32),
        mesh=mesh,
        compiler_params=pltpu.CompilerParams(needs_layout_passes=False),
        scratch_types=dict(
            hi=pltpu.VMEM((NCHUNK, CHUNK), jnp.int32),
            ri=pltpu.VMEM((NCHUNK, CHUNK), jnp.int32),
            ti=pltpu.VMEM((NCHUNK, CHUNK), jnp.int32),
            hs=pltpu.VMEM((BPW,), jnp.int32),
            rs=pltpu.VMEM((BPW,), jnp.int32),
            ts=pltpu.VMEM((BPW,), jnp.int32),
            hrow=pltpu.VMEM((2, CHUNK, ROWW), jnp.float32),
            rrow=pltpu.VMEM((2, CHUNK, ROWW), jnp.float32),
            trow=pltpu.VMEM((2, CHUNK, ROWW), jnp.float32),
            out_v=pltpu.VMEM((BPW,), jnp.float32),
            sem_h=pltpu.SemaphoreType.DMA((2,)),
            sem_r=pltpu.SemaphoreType.DMA((2,)),
            sem_t=pltpu.SemaphoreType.DMA((2,)),
        ),
    )
    def run(e_hbm, rel_hbm, hi_hbm, ri_hbm, ti_hbm, hs_hbm, rs_hbm, ts_hbm,
            out_hbm, hi, ri, ti, hs, rs, ts, hrow, rrow, trow, out_v,
            sem_h, sem_r, sem_t):
        wid = lax.axis_index("s") * NC + lax.axis_index("c")
        pltpu.sync_copy(hi_hbm.at[wid], hi)
        pltpu.sync_copy(ri_hbm.at[wid], ri)
        pltpu.sync_copy(ti_hbm.at[wid], ti)
        pltpu.sync_copy(hs_hbm.at[pl.ds(wid * BPW, BPW)], hs)
        pltpu.sync_copy(rs_hbm.at[pl.ds(wid * BPW, BPW)], rs)
        pltpu.sync_copy(ts_hbm.at[pl.ds(wid * BPW, BPW)], ts)

        def fire(c, b):
            pltpu.async_copy(e_hbm.at[hi.at[c]], hrow.at[b], sem_h.at[b])
            pltpu.async_copy(rel_hbm.at[ri.at[c]], rrow.at[b], sem_r.at[b])
            pltpu.async_copy(e_hbm.at[ti.at[c]], trow.at[b], sem_t.at[b])

        fire(0, 0)

        def chunk_body(c, carry):
            b = lax.rem(c, 2)

            @pl.when(c + 1 < NCHUNK)
            def _():
                fire(c + 1, 1 - b)

            dummy = e_hbm.at[pl.ds(0, CHUNK)]
            pltpu.make_async_copy(dummy, hrow.at[b], sem_h.at[b]).wait()
            pltpu.make_async_copy(dummy, rrow.at[b], sem_r.at[b]).wait()
            pltpu.make_async_copy(dummy, trow.at[b], sem_t.at[b]).wait()

            def grp(g, carry2):
                base = c * CHUNK + g * LANES
                rows = lax.iota(jnp.int32, LANES) + g * LANES
                sh = hs[pl.ds(base, LANES)]
                sr = rs[pl.ds(base, LANES)]
                st = ts[pl.ds(base, LANES)]
                acc = jnp.zeros((LANES,), jnp.float32)
                for d in range(DIM):
                    hc = plsc.load_gather(hrow.at[b], [rows, sh + d])
                    rc = plsc.load_gather(rrow.at[b], [rows, sr + d])
                    tc = plsc.load_gather(trow.at[b], [rows, st + d])
                    acc = acc + jnp.abs(hc + rc - tc)
                out_v[pl.ds(base, LANES)] = acc
                return carry2

            lax.fori_loop(0, GRP, grp, 0)
            return carry

        lax.fori_loop(0, NCHUNK, chunk_body, 0)
        pltpu.sync_copy(out_v, out_hbm.at[pl.ds(wid * BPW, BPW)])

    return run(e4, r4, h_idx4, r_idx4, t_idx4, h_sub, r_sub, t_sub)


def kernel(pos_triples, neg_triples, e_weight, r_weight):
    e4 = _pack_rows_tc(e_weight.T)
    r4 = _pack_rows_tc(r_weight.T)
    trip = jnp.concatenate(
        [pos_triples.astype(jnp.int32), neg_triples.astype(jnp.int32)], axis=1)
    row = (trip // TBLK) * SUB + jnp.remainder(trip, SUB)
    idx4 = row.reshape(3, NW, NCHUNK, CHUNK)
    sub = (jnp.remainder(trip // SUB, PACK) * DIM).reshape(3, NW * BPW)
    out = _gather_sc(e4, r4, idx4[0], idx4[1], idx4[2],
                     sub[0], sub[1], sub[2])
    return (out[:BATCH], out[BATCH:])
